# baseline (speedup 1.0000x reference)
import functools

import jax
import jax.numpy as jnp
from jax import lax
from jax.experimental import pallas as pl
from jax.experimental.pallas import tpu as pltpu
from jax.experimental.pallas import tpu_sc as plsc

N = 10000
E = 320000
IN_CH = 128
HID = 16
OUT_CH = 16
HEADS = 8

NS = 16
NW = 2 * NS
NPAD = 10112
B = 128
EPW = 10368
ETOT = EPW * NW
D1 = HEADS * HID + 16
D2 = OUT_CH + 16
BLK = 128


def _chunks(total, step):
    out = []
    off = 0
    while off < total:
        out.append((off, min(step, total - off)))
        off += step
    return out


def _make_edge_pass(D, nh, B):
    HOFF = nh * 16
    RPT = NPAD // NS
    NB = EPW // B
    assert EPW % B == 0 and NB % 2 == 0 and B % 16 == 0

    mesh = plsc.VectorSubcoreMesh(core_axis_name="c", subcore_axis_name="s")

    @functools.partial(
        pl.kernel,
        out_type=jax.ShapeDtypeStruct((2, NPAD, D), jnp.float32),
        mesh=mesh,
        scratch_types=[
            pltpu.VMEM((2, B), jnp.int32),
            pltpu.VMEM((2, B), jnp.int32),
            pltpu.VMEM((B,), jnp.int32),
            pltpu.VMEM((B,), jnp.int32),
            pltpu.VMEM((B, D), jnp.float32),
            pltpu.VMEM((B, D), jnp.float32),
            pltpu.VMEM((B, 16), jnp.float32),
            pltpu.VMEM((B, 16), jnp.float32),
            pltpu.VMEM((B, D), jnp.float32),
            pltpu.VMEM((B, D), jnp.float32),
            pltpu.VMEM_SHARED((NPAD, D), jnp.float32),
            pltpu.SemaphoreType.DMA,
            pltpu.SemaphoreType.DMA,
            pltpu.SemaphoreType.DMA,
            pltpu.SemaphoreType.DMA,
            pltpu.SemaphoreType.DMA,
            pltpu.SemaphoreType.DMA,
            pltpu.SemaphoreType.DMA,
            pltpu.SemaphoreType.DMA,
        ],
        compiler_params=pltpu.CompilerParams(use_tc_tiling_on_sc=False),
    )
    def edge_pass(g_hbm, a_hbm, sd_hbm, out_hbm,
                  idx0, idx1, isc0, isc1, gr0, gr1, ar0, ar1, ms0, ms1, acc,
                  sg0, sg1, sa0, sa1, ss0, ss1, si0, si1):
        idx = (idx0, idx1)
        isc = (isc0, isc1)
        grows = (gr0, gr1)
        arows = (ar0, ar1)
        msg = (ms0, ms1)
        sem_g = (sg0, sg1)
        sem_a = (sa0, sa1)
        sem_s = (ss0, ss1)
        sem_i = (si0, si1)

        c = lax.axis_index("c")
        s = lax.axis_index("s")
        wid = c * NS + s

        def zrow(r, carry):
            for j in range(D // 16):
                ms0[r, pl.ds(16 * j, 16)] = jnp.zeros((16,), jnp.float32)
            return carry
        lax.fori_loop(0, B, zrow, 0)
        for off, nrows in _chunks(RPT, B):
            pltpu.sync_copy(ms0.at[pl.ds(0, nrows)],
                            acc.at[pl.ds(s * RPT + off, nrows)])
        plsc.subcore_barrier()

        def iload(b, p):
            pltpu.async_copy(sd_hbm.at[wid, b], idx[p], sem_i[p])

        def iwait(p):
            pltpu.make_async_copy(sd_hbm.at[wid, 0], idx[p], sem_i[p]).wait()

        def gissue(p):
            pltpu.async_copy(g_hbm.at[idx[p].at[0]], grows[p], sem_g[p])
            pltpu.async_copy(a_hbm.at[idx[p].at[1]], arows[p], sem_a[p])

        def gwait(p):
            pltpu.make_async_copy(g_hbm.at[idx[p].at[0]], grows[p], sem_g[p]).wait()
            pltpu.make_async_copy(a_hbm.at[idx[p].at[1]], arows[p], sem_a[p]).wait()

        def swait(p):
            pltpu.make_async_copy(msg[p], acc.at[isc[p]], sem_s[p]).wait()

        def compute(p):
            gr, ar, ms = grows[p], arows[p], msg[p]

            def edge(e, ecarry):
                av = gr[e, pl.ds(HOFF, 16)]
                dv = ar[e, pl.ds(0, 16)]
                t = av + dv
                t = jnp.where(t >= 0.0, t, 0.2 * t)
                w16 = jnp.exp(t)
                ms[e, pl.ds(HOFF, 16)] = w16
                if nh == 1:
                    ms[e, pl.ds(0, 16)] = w16 * gr[e, pl.ds(0, 16)]
                else:
                    for j in range(nh):
                        wj = w16[j]
                        ms[e, pl.ds(16 * j, 16)] = wj * gr[e, pl.ds(16 * j, 16)]
                return ecarry
            lax.fori_loop(0, B, edge, 0, unroll=2)

        def proc(b, p, do_ws, do_gnext, do_inext):
            q = 1 - p
            if do_ws:
                swait(p)
            if do_gnext:
                iwait(q)
                gissue(q)
            gwait(p)
            for j in range(B // 16):
                isc[p][pl.ds(16 * j, 16)] = idx[p][1, pl.ds(16 * j, 16)]
            if do_inext:
                iload(b + 2, p)
            compute(p)
            pltpu.async_copy(msg[p], acc.at[isc[p]], sem_s[p], add=True)

        pltpu.sync_copy(sd_hbm.at[wid, 0], idx[0])
        gissue(0)
        iload(1, 1)
        proc(0, 0, False, True, True)
        proc(1, 1, False, True, True)

        def pairbody(k, carry):
            proc(2 * k, 0, True, True, True)
            proc(2 * k + 1, 1, True, True, True)
            return carry
        lax.fori_loop(1, NB // 2 - 1, pairbody, 0)

        proc(NB - 2, 0, True, True, False)
        proc(NB - 1, 1, True, False, False)
        swait(0)
        swait(1)

        plsc.subcore_barrier()
        for off, nrows in _chunks(RPT, B):
            r0 = s * RPT + off
            pltpu.sync_copy(acc.at[pl.ds(r0, nrows)],
                            out_hbm.at[c, pl.ds(r0, nrows)])

    return edge_pass


B1 = 64
B2 = 96
_edge_pass_l1 = _make_edge_pass(D1, HEADS, B1)
_edge_pass_l2 = _make_edge_pass(D2, 1, B2)


def _prep1_body(x_ref, w_ref, as_ref, ad_ref, g_ref, a_ref):
    h = jnp.dot(x_ref[:], w_ref[:], preferred_element_type=jnp.float32)
    asrc16 = jnp.dot(h, as_ref[:], preferred_element_type=jnp.float32)
    adst16 = jnp.dot(h, ad_ref[:], preferred_element_type=jnp.float32)
    g_ref[:] = jnp.concatenate([h, asrc16], axis=1)
    a_ref[:] = adst16


def _mid_body(acc0_ref, acc1_ref, w2_ref, as2_ref, ad2_ref, r_ref, b1_ref,
              g_ref, a_ref):
    acc = acc0_ref[:] + acc1_ref[:]
    wsum = acc[:, IN_CH:IN_CH + HEADS]
    recip = 1.0 / jnp.maximum(wsum, 1e-30)
    rep = jnp.dot(recip, r_ref[:], preferred_element_type=jnp.float32)
    hmid = jnp.maximum(acc[:, :IN_CH] * rep + b1_ref[:], 0.0)
    h2 = jnp.dot(hmid, w2_ref[:], preferred_element_type=jnp.float32)
    asrc2 = jnp.sum(h2 * as2_ref[:], axis=1, keepdims=True)
    adst2 = jnp.sum(h2 * ad2_ref[:], axis=1, keepdims=True)
    g_ref[:] = jnp.concatenate(
        [h2, jnp.broadcast_to(asrc2, (h2.shape[0], 16))], axis=1)
    a_ref[:] = jnp.broadcast_to(adst2, (h2.shape[0], 16))


def _final_body(acc0_ref, acc1_ref, b2_ref, o_ref):
    acc = acc0_ref[:] + acc1_ref[:]
    den = jnp.maximum(acc[:, OUT_CH:2 * OUT_CH], 1e-30)
    o = acc[:, :OUT_CH] / den + b2_ref[:]
    m = jnp.max(o, axis=1, keepdims=True)
    z = o - m
    o_ref[:] = z - jnp.log(jnp.sum(jnp.exp(z), axis=1, keepdims=True))


def _row_spec(d):
    return pl.BlockSpec((BLK, d), lambda i: (i, 0))


def _full_spec(r, d):
    return pl.BlockSpec((r, d), lambda i: (0, 0))


def kernel(x, edge_index, W1, a1_src, a1_dst, b1, W2, a2_src, a2_dst, b2):
    f32 = jnp.float32
    src = edge_index[0].astype(jnp.int32)
    dst = edge_index[1].astype(jnp.int32)
    loop = jnp.arange(N, dtype=jnp.int32)
    padi = jnp.full((ETOT - E - N,), N, jnp.int32)
    src_all = jnp.concatenate([src, loop, padi])
    dst_all = jnp.concatenate([dst, loop, padi])
    sd1 = jnp.stack([src_all.reshape(NW, EPW // B1, B1),
                     dst_all.reshape(NW, EPW // B1, B1)], axis=2)
    sd2 = jnp.stack([src_all.reshape(NW, EPW // B2, B2),
                     dst_all.reshape(NW, EPW // B2, B2)], axis=2)

    xpad = jnp.pad(x.astype(f32), ((0, NPAD - N), (0, 0)))
    eye = jnp.eye(HEADS, dtype=f32)
    As1 = (a1_src.astype(f32)[:, :, None] * eye[:, None, :]).reshape(IN_CH, HEADS)
    Ad1 = (a1_dst.astype(f32)[:, :, None] * eye[:, None, :]).reshape(IN_CH, HEADS)
    As1d = jnp.concatenate([As1, As1], axis=1)
    Ad1d = jnp.concatenate([Ad1, Ad1], axis=1)
    R = jnp.repeat(eye, HID, axis=1)
    b1r = b1.astype(f32).reshape(1, IN_CH)
    b2r = b2.astype(f32).reshape(1, OUT_CH)
    as2 = a2_src.astype(f32).reshape(1, OUT_CH)
    ad2 = a2_dst.astype(f32).reshape(1, OUT_CH)

    grid = (NPAD // BLK,)

    G1, A1 = pl.pallas_call(
        _prep1_body,
        grid=grid,
        in_specs=[
            _row_spec(IN_CH),
            _full_spec(IN_CH, IN_CH),
            _full_spec(IN_CH, 16),
            _full_spec(IN_CH, 16),
        ],
        out_specs=[_row_spec(D1), _row_spec(16)],
        out_shape=[
            jax.ShapeDtypeStruct((NPAD, D1), f32),
            jax.ShapeDtypeStruct((NPAD, 16), f32),
        ],
    )(xpad, W1.astype(f32), As1d, Ad1d)

    acc1 = _edge_pass_l1(G1, A1, sd1)

    G2, A2 = pl.pallas_call(
        _mid_body,
        grid=grid,
        in_specs=[
            _row_spec(D1),
            _row_spec(D1),
            _full_spec(IN_CH, OUT_CH),
            _full_spec(1, OUT_CH),
            _full_spec(1, OUT_CH),
            _full_spec(HEADS, IN_CH),
            _full_spec(1, IN_CH),
        ],
        out_specs=[_row_spec(D2), _row_spec(16)],
        out_shape=[
            jax.ShapeDtypeStruct((NPAD, D2), f32),
            jax.ShapeDtypeStruct((NPAD, 16), f32),
        ],
    )(acc1[0], acc1[1], W2.astype(f32), as2, ad2, R, b1r)

    acc2 = _edge_pass_l2(G2, A2, sd2)

    o = pl.pallas_call(
        _final_body,
        grid=grid,
        in_specs=[_row_spec(D2), _row_spec(D2), _full_spec(1, OUT_CH)],
        out_specs=_row_spec(OUT_CH),
        out_shape=jax.ShapeDtypeStruct((NPAD, OUT_CH), f32),
    )(acc2[0], acc2[1], b2r)

    return o[:N]


# speedup vs baseline: 1.4188x; 1.4188x over previous
import functools

import jax
import jax.numpy as jnp
from jax import lax
from jax.experimental import pallas as pl
from jax.experimental.pallas import tpu as pltpu
from jax.experimental.pallas import tpu_sc as plsc

N = 10000
E = 320000
IN_CH = 128
HID = 16
OUT_CH = 16
HEADS = 8

NS = 16
NW = 2 * NS
NPAD = 10112
B = 128
EPW = 10368
ETOT = EPW * NW
D1 = HEADS * HID + 16
D2 = OUT_CH + 16
BLK = 128


def _chunks(total, step):
    out = []
    off = 0
    while off < total:
        out.append((off, min(step, total - off)))
        off += step
    return out


def _make_edge_pass(D, nh, B):
    HOFF = nh * 16
    RPT = NPAD // NS
    NB = EPW // B
    assert EPW % B == 0 and NB % 2 == 0 and B % 16 == 0

    mesh = plsc.VectorSubcoreMesh(core_axis_name="c", subcore_axis_name="s")

    @functools.partial(
        pl.kernel,
        out_type=jax.ShapeDtypeStruct((2, NPAD, D), jnp.float32),
        mesh=mesh,
        scratch_types=[
            pltpu.VMEM((2, B), jnp.int32),
            pltpu.VMEM((2, B), jnp.int32),
            pltpu.VMEM((B,), jnp.int32),
            pltpu.VMEM((B,), jnp.int32),
            pltpu.VMEM((B, D), jnp.float32),
            pltpu.VMEM((B, D), jnp.float32),
            pltpu.VMEM((B, 16), jnp.float32),
            pltpu.VMEM((B, 16), jnp.float32),
            pltpu.VMEM((B, D), jnp.float32),
            pltpu.VMEM((B, D), jnp.float32),
            pltpu.VMEM_SHARED((NPAD, D), jnp.float32),
            pltpu.SemaphoreType.DMA,
            pltpu.SemaphoreType.DMA,
            pltpu.SemaphoreType.DMA,
            pltpu.SemaphoreType.DMA,
            pltpu.SemaphoreType.DMA,
            pltpu.SemaphoreType.DMA,
            pltpu.SemaphoreType.DMA,
            pltpu.SemaphoreType.DMA,
        ],
        compiler_params=pltpu.CompilerParams(use_tc_tiling_on_sc=False),
    )
    def edge_pass(g_hbm, a_hbm, sd_hbm, out_hbm,
                  idx0, idx1, isc0, isc1, gr0, gr1, ar0, ar1, ms0, ms1, acc,
                  sg0, sg1, sa0, sa1, ss0, ss1, si0, si1):
        idx = (idx0, idx1)
        isc = (isc0, isc1)
        grows = (gr0, gr1)
        arows = (ar0, ar1)
        msg = (ms0, ms1)
        sem_g = (sg0, sg1)
        sem_a = (sa0, sa1)
        sem_s = (ss0, ss1)
        sem_i = (si0, si1)

        c = lax.axis_index("c")
        s = lax.axis_index("s")
        wid = c * NS + s

        def zrow(r, carry):
            for j in range(D // 16):
                ms0[r, pl.ds(16 * j, 16)] = jnp.zeros((16,), jnp.float32)
            return carry
        lax.fori_loop(0, B, zrow, 0)
        for off, nrows in _chunks(RPT, B):
            pltpu.sync_copy(ms0.at[pl.ds(0, nrows)],
                            acc.at[pl.ds(s * RPT + off, nrows)])
        plsc.subcore_barrier()

        def iload(b, p):
            pltpu.async_copy(sd_hbm.at[wid, b], idx[p], sem_i[p])

        def iwait(p):
            pltpu.make_async_copy(sd_hbm.at[wid, 0], idx[p], sem_i[p]).wait()

        def gissue(p):
            pltpu.async_copy(g_hbm.at[idx[p].at[0]], grows[p], sem_g[p])
            pltpu.async_copy(a_hbm.at[idx[p].at[1]], arows[p], sem_a[p])

        def gwait(p):
            pltpu.make_async_copy(g_hbm.at[idx[p].at[0]], grows[p], sem_g[p]).wait()
            pltpu.make_async_copy(a_hbm.at[idx[p].at[1]], arows[p], sem_a[p]).wait()

        def swait(p):
            pltpu.make_async_copy(msg[p], acc.at[isc[p]], sem_s[p]).wait()

        def compute(p):
            gr, ar, ms = grows[p], arows[p], msg[p]

            def edge(e, ecarry):
                av = gr[e, pl.ds(HOFF, 16)]
                dv = ar[e, pl.ds(0, 16)]
                t = av + dv
                t = jnp.where(t >= 0.0, t, 0.2 * t)
                w16 = jnp.exp(t)
                ms[e, pl.ds(HOFF, 16)] = w16
                if nh == 1:
                    ms[e, pl.ds(0, 16)] = w16 * gr[e, pl.ds(0, 16)]
                else:
                    for j in range(nh):
                        wj = w16[j]
                        ms[e, pl.ds(16 * j, 16)] = wj * gr[e, pl.ds(16 * j, 16)]
                return ecarry
            lax.fori_loop(0, B, edge, 0)

        def proc(b, p, do_ws, do_gnext, do_inext):
            q = 1 - p
            if do_ws:
                swait(p)
            if do_gnext:
                iwait(q)
                gissue(q)
            gwait(p)
            for j in range(B // 16):
                isc[p][pl.ds(16 * j, 16)] = idx[p][1, pl.ds(16 * j, 16)]
            if do_inext:
                iload(b + 2, p)
            compute(p)
            pltpu.async_copy(msg[p], acc.at[isc[p]], sem_s[p], add=True)

        pltpu.sync_copy(sd_hbm.at[wid, 0], idx[0])
        gissue(0)
        iload(1, 1)
        proc(0, 0, False, True, True)
        proc(1, 1, False, True, True)

        def pairbody(k, carry):
            proc(2 * k, 0, True, True, True)
            proc(2 * k + 1, 1, True, True, True)
            return carry
        lax.fori_loop(1, NB // 2 - 1, pairbody, 0)

        proc(NB - 2, 0, True, True, False)
        proc(NB - 1, 1, True, False, False)
        swait(0)
        swait(1)

        plsc.subcore_barrier()
        for off, nrows in _chunks(RPT, B):
            r0 = s * RPT + off
            pltpu.sync_copy(acc.at[pl.ds(r0, nrows)],
                            out_hbm.at[c, pl.ds(r0, nrows)])

    return edge_pass


B1 = 64
B2 = 96
_edge_pass_l1 = _make_edge_pass(D1, HEADS, B1)
_edge_pass_l2 = _make_edge_pass(D2, 1, B2)


def _prep1_body(x_ref, w_ref, as_ref, ad_ref, g_ref, a_ref):
    h = jnp.dot(x_ref[:], w_ref[:], preferred_element_type=jnp.float32)
    asrc16 = jnp.dot(h, as_ref[:], preferred_element_type=jnp.float32)
    adst16 = jnp.dot(h, ad_ref[:], preferred_element_type=jnp.float32)
    g_ref[:] = jnp.concatenate([h, asrc16], axis=1)
    a_ref[:] = adst16


def _mid_body(acc0_ref, acc1_ref, w2_ref, as2_ref, ad2_ref, r_ref, b1_ref,
              g_ref, a_ref):
    acc = acc0_ref[:] + acc1_ref[:]
    wsum = acc[:, IN_CH:IN_CH + HEADS]
    recip = 1.0 / jnp.maximum(wsum, 1e-30)
    rep = jnp.dot(recip, r_ref[:], preferred_element_type=jnp.float32)
    hmid = jnp.maximum(acc[:, :IN_CH] * rep + b1_ref[:], 0.0)
    h2 = jnp.dot(hmid, w2_ref[:], preferred_element_type=jnp.float32)
    asrc2 = jnp.sum(h2 * as2_ref[:], axis=1, keepdims=True)
    adst2 = jnp.sum(h2 * ad2_ref[:], axis=1, keepdims=True)
    g_ref[:] = jnp.concatenate(
        [h2, jnp.broadcast_to(asrc2, (h2.shape[0], 16))], axis=1)
    a_ref[:] = jnp.broadcast_to(adst2, (h2.shape[0], 16))


def _final_body(acc0_ref, acc1_ref, b2_ref, o_ref):
    acc = acc0_ref[:] + acc1_ref[:]
    den = jnp.maximum(acc[:, OUT_CH:2 * OUT_CH], 1e-30)
    o = acc[:, :OUT_CH] / den + b2_ref[:]
    m = jnp.max(o, axis=1, keepdims=True)
    z = o - m
    o_ref[:] = z - jnp.log(jnp.sum(jnp.exp(z), axis=1, keepdims=True))


def _row_spec(d):
    return pl.BlockSpec((BLK, d), lambda i: (i, 0))


def _full_spec(r, d):
    return pl.BlockSpec((r, d), lambda i: (0, 0))


def kernel(x, edge_index, W1, a1_src, a1_dst, b1, W2, a2_src, a2_dst, b2):
    f32 = jnp.float32
    src = edge_index[0].astype(jnp.int32)
    dst = edge_index[1].astype(jnp.int32)
    loop = jnp.arange(N, dtype=jnp.int32)
    padi = jnp.full((ETOT - E - N,), N, jnp.int32)
    src_all = jnp.concatenate([src, loop, padi])
    dst_all = jnp.concatenate([dst, loop, padi])
    sd1 = jnp.stack([src_all.reshape(NW, EPW // B1, B1),
                     dst_all.reshape(NW, EPW // B1, B1)], axis=2)
    sd2 = jnp.stack([src_all.reshape(NW, EPW // B2, B2),
                     dst_all.reshape(NW, EPW // B2, B2)], axis=2)

    xpad = jnp.pad(x.astype(f32), ((0, NPAD - N), (0, 0)))
    eye = jnp.eye(HEADS, dtype=f32)
    As1 = (a1_src.astype(f32)[:, :, None] * eye[:, None, :]).reshape(IN_CH, HEADS)
    Ad1 = (a1_dst.astype(f32)[:, :, None] * eye[:, None, :]).reshape(IN_CH, HEADS)
    As1d = jnp.concatenate([As1, As1], axis=1)
    Ad1d = jnp.concatenate([Ad1, Ad1], axis=1)
    R = jnp.repeat(eye, HID, axis=1)
    b1r = b1.astype(f32).reshape(1, IN_CH)
    b2r = b2.astype(f32).reshape(1, OUT_CH)
    as2 = a2_src.astype(f32).reshape(1, OUT_CH)
    ad2 = a2_dst.astype(f32).reshape(1, OUT_CH)

    grid = (NPAD // BLK,)

    G1, A1 = pl.pallas_call(
        _prep1_body,
        grid=grid,
        in_specs=[
            _row_spec(IN_CH),
            _full_spec(IN_CH, IN_CH),
            _full_spec(IN_CH, 16),
            _full_spec(IN_CH, 16),
        ],
        out_specs=[_row_spec(D1), _row_spec(16)],
        out_shape=[
            jax.ShapeDtypeStruct((NPAD, D1), f32),
            jax.ShapeDtypeStruct((NPAD, 16), f32),
        ],
    )(xpad, W1.astype(f32), As1d, Ad1d)

    acc1 = _edge_pass_l1(G1, A1, sd1)

    G2, A2 = pl.pallas_call(
        _mid_body,
        grid=grid,
        in_specs=[
            _row_spec(D1),
            _row_spec(D1),
            _full_spec(IN_CH, OUT_CH),
            _full_spec(1, OUT_CH),
            _full_spec(1, OUT_CH),
            _full_spec(HEADS, IN_CH),
            _full_spec(1, IN_CH),
        ],
        out_specs=[_row_spec(D2), _row_spec(16)],
        out_shape=[
            jax.ShapeDtypeStruct((NPAD, D2), f32),
            jax.ShapeDtypeStruct((NPAD, 16), f32),
        ],
    )(acc1[0], acc1[1], W2.astype(f32), as2, ad2, R, b1r)

    acc2 = _edge_pass_l2(G2, A2, sd2)

    o = pl.pallas_call(
        _final_body,
        grid=grid,
        in_specs=[_row_spec(D2), _row_spec(D2), _full_spec(1, OUT_CH)],
        out_specs=_row_spec(OUT_CH),
        out_shape=jax.ShapeDtypeStruct((NPAD, OUT_CH), f32),
    )(acc2[0], acc2[1], b2r)

    return o[:N]


# speedup vs baseline: 1.4208x; 1.0014x over previous
import functools

import jax
import jax.numpy as jnp
from jax import lax
from jax.experimental import pallas as pl
from jax.experimental.pallas import tpu as pltpu
from jax.experimental.pallas import tpu_sc as plsc

N = 10000
E = 320000
IN_CH = 128
HID = 16
OUT_CH = 16
HEADS = 8

NS = 16
NW = 2 * NS
NPAD = 10112
B = 128
EPW = 10368
ETOT = EPW * NW
D1 = HEADS * HID + 16
D2 = OUT_CH + 16
BLK = 128


def _chunks(total, step):
    out = []
    off = 0
    while off < total:
        out.append((off, min(step, total - off)))
        off += step
    return out


def _make_edge_pass(D, nh, B):
    HOFF = nh * 16
    RPT = NPAD // NS
    NB = EPW // B
    assert EPW % B == 0 and NB % 2 == 0 and B % 16 == 0

    mesh = plsc.VectorSubcoreMesh(core_axis_name="c", subcore_axis_name="s")

    @functools.partial(
        pl.kernel,
        out_type=jax.ShapeDtypeStruct((2, NPAD, D), jnp.float32),
        mesh=mesh,
        scratch_types=[
            pltpu.VMEM((2, B), jnp.int32),
            pltpu.VMEM((2, B), jnp.int32),
            pltpu.VMEM((B,), jnp.int32),
            pltpu.VMEM((B,), jnp.int32),
            pltpu.VMEM((B, D), jnp.float32),
            pltpu.VMEM((B, D), jnp.float32),
            pltpu.VMEM((B, 16), jnp.float32),
            pltpu.VMEM((B, 16), jnp.float32),
            pltpu.VMEM((B, D), jnp.float32),
            pltpu.VMEM((B, D), jnp.float32),
            pltpu.VMEM_SHARED((NPAD, D), jnp.float32),
            pltpu.SemaphoreType.DMA,
            pltpu.SemaphoreType.DMA,
            pltpu.SemaphoreType.DMA,
            pltpu.SemaphoreType.DMA,
            pltpu.SemaphoreType.DMA,
            pltpu.SemaphoreType.DMA,
            pltpu.SemaphoreType.DMA,
            pltpu.SemaphoreType.DMA,
        ],
        compiler_params=pltpu.CompilerParams(use_tc_tiling_on_sc=False),
    )
    def edge_pass(g_hbm, a_hbm, sd_hbm, out_hbm,
                  idx0, idx1, isc0, isc1, gr0, gr1, ar0, ar1, ms0, ms1, acc,
                  sg0, sg1, sa0, sa1, ss0, ss1, si0, si1):
        idx = (idx0, idx1)
        isc = (isc0, isc1)
        grows = (gr0, gr1)
        arows = (ar0, ar1)
        msg = (ms0, ms1)
        sem_g = (sg0, sg1)
        sem_a = (sa0, sa1)
        sem_s = (ss0, ss1)
        sem_i = (si0, si1)

        c = lax.axis_index("c")
        s = lax.axis_index("s")
        wid = c * NS + s

        def iload(b, p):
            pltpu.async_copy(sd_hbm.at[wid, b], idx[p], sem_i[p])

        def iwait(p):
            pltpu.make_async_copy(sd_hbm.at[wid, 0], idx[p], sem_i[p]).wait()

        def gissue(p):
            pltpu.async_copy(g_hbm.at[idx[p].at[0]], grows[p], sem_g[p])
            pltpu.async_copy(a_hbm.at[idx[p].at[1]], arows[p], sem_a[p])

        def gwait(p):
            pltpu.make_async_copy(g_hbm.at[idx[p].at[0]], grows[p], sem_g[p]).wait()
            pltpu.make_async_copy(a_hbm.at[idx[p].at[1]], arows[p], sem_a[p]).wait()

        def swait(p):
            pltpu.make_async_copy(msg[p], acc.at[isc[p]], sem_s[p]).wait()

        def compute(p):
            gr, ar, ms = grows[p], arows[p], msg[p]

            def edge(e, ecarry):
                av = gr[e, pl.ds(HOFF, 16)]
                dv = ar[e, pl.ds(0, 16)]
                t = av + dv
                t = jnp.where(t >= 0.0, t, 0.2 * t)
                w16 = jnp.exp(t)
                ms[e, pl.ds(HOFF, 16)] = w16
                if nh == 1:
                    ms[e, pl.ds(0, 16)] = w16 * gr[e, pl.ds(0, 16)]
                else:
                    for j in range(nh):
                        wj = w16[j]
                        ms[e, pl.ds(16 * j, 16)] = wj * gr[e, pl.ds(16 * j, 16)]
                return ecarry
            lax.fori_loop(0, B, edge, 0)

        def proc(b, p, do_ws, do_gnext, do_inext):
            q = 1 - p
            if do_ws:
                swait(p)
            if do_gnext:
                iwait(q)
                gissue(q)
            gwait(p)
            for j in range(B // 16):
                isc[p][pl.ds(16 * j, 16)] = idx[p][1, pl.ds(16 * j, 16)]
            if do_inext:
                iload(b + 2, p)
            compute(p)
            pltpu.async_copy(msg[p], acc.at[isc[p]], sem_s[p], add=True)

        pltpu.sync_copy(sd_hbm.at[wid, 0], idx[0])
        gissue(0)
        iload(1, 1)

        def zrow(r, carry):
            for j in range(D // 16):
                ms0[r, pl.ds(16 * j, 16)] = jnp.zeros((16,), jnp.float32)
            return carry
        lax.fori_loop(0, B, zrow, 0)
        for off, nrows in _chunks(RPT, B):
            pltpu.sync_copy(ms0.at[pl.ds(0, nrows)],
                            acc.at[pl.ds(s * RPT + off, nrows)])
        plsc.subcore_barrier()

        proc(0, 0, False, True, True)
        proc(1, 1, False, True, True)

        def pairbody(k, carry):
            proc(2 * k, 0, True, True, True)
            proc(2 * k + 1, 1, True, True, True)
            return carry
        lax.fori_loop(1, NB // 2 - 1, pairbody, 0)

        proc(NB - 2, 0, True, True, False)
        proc(NB - 1, 1, True, False, False)
        swait(0)
        swait(1)

        plsc.subcore_barrier()
        for off, nrows in _chunks(RPT, B):
            r0 = s * RPT + off
            pltpu.sync_copy(acc.at[pl.ds(r0, nrows)],
                            out_hbm.at[c, pl.ds(r0, nrows)])

    return edge_pass


B1 = 64
B2 = 96
_edge_pass_l1 = _make_edge_pass(D1, HEADS, B1)
_edge_pass_l2 = _make_edge_pass(D2, 1, B2)


def _prep1_body(x_ref, w_ref, as_ref, ad_ref, g_ref, a_ref):
    h = jnp.dot(x_ref[:], w_ref[:], preferred_element_type=jnp.float32)
    asrc16 = jnp.dot(h, as_ref[:], preferred_element_type=jnp.float32)
    adst16 = jnp.dot(h, ad_ref[:], preferred_element_type=jnp.float32)
    g_ref[:] = jnp.concatenate([h, asrc16], axis=1)
    a_ref[:] = adst16


def _mid_body(acc0_ref, acc1_ref, w2_ref, as2_ref, ad2_ref, r_ref, b1_ref,
              g_ref, a_ref):
    acc = acc0_ref[:] + acc1_ref[:]
    wsum = acc[:, IN_CH:IN_CH + HEADS]
    recip = 1.0 / jnp.maximum(wsum, 1e-30)
    rep = jnp.dot(recip, r_ref[:], preferred_element_type=jnp.float32)
    hmid = jnp.maximum(acc[:, :IN_CH] * rep + b1_ref[:], 0.0)
    h2 = jnp.dot(hmid, w2_ref[:], preferred_element_type=jnp.float32)
    asrc2 = jnp.sum(h2 * as2_ref[:], axis=1, keepdims=True)
    adst2 = jnp.sum(h2 * ad2_ref[:], axis=1, keepdims=True)
    g_ref[:] = jnp.concatenate(
        [h2, jnp.broadcast_to(asrc2, (h2.shape[0], 16))], axis=1)
    a_ref[:] = jnp.broadcast_to(adst2, (h2.shape[0], 16))


def _final_body(acc0_ref, acc1_ref, b2_ref, o_ref):
    acc = acc0_ref[:] + acc1_ref[:]
    den = jnp.maximum(acc[:, OUT_CH:2 * OUT_CH], 1e-30)
    o = acc[:, :OUT_CH] / den + b2_ref[:]
    m = jnp.max(o, axis=1, keepdims=True)
    z = o - m
    o_ref[:] = z - jnp.log(jnp.sum(jnp.exp(z), axis=1, keepdims=True))


def _row_spec(d):
    return pl.BlockSpec((BLK, d), lambda i: (i, 0))


def _full_spec(r, d):
    return pl.BlockSpec((r, d), lambda i: (0, 0))


def kernel(x, edge_index, W1, a1_src, a1_dst, b1, W2, a2_src, a2_dst, b2):
    f32 = jnp.float32
    src = edge_index[0].astype(jnp.int32)
    dst = edge_index[1].astype(jnp.int32)
    loop = jnp.arange(N, dtype=jnp.int32)
    padi = jnp.full((ETOT - E - N,), N, jnp.int32)
    src_all = jnp.concatenate([src, loop, padi])
    dst_all = jnp.concatenate([dst, loop, padi])
    sd1 = jnp.stack([src_all.reshape(NW, EPW // B1, B1),
                     dst_all.reshape(NW, EPW // B1, B1)], axis=2)
    sd2 = jnp.stack([src_all.reshape(NW, EPW // B2, B2),
                     dst_all.reshape(NW, EPW // B2, B2)], axis=2)

    xpad = jnp.pad(x.astype(f32), ((0, NPAD - N), (0, 0)))
    eye = jnp.eye(HEADS, dtype=f32)
    As1 = (a1_src.astype(f32)[:, :, None] * eye[:, None, :]).reshape(IN_CH, HEADS)
    Ad1 = (a1_dst.astype(f32)[:, :, None] * eye[:, None, :]).reshape(IN_CH, HEADS)
    As1d = jnp.concatenate([As1, As1], axis=1)
    Ad1d = jnp.concatenate([Ad1, Ad1], axis=1)
    R = jnp.repeat(eye, HID, axis=1)
    b1r = b1.astype(f32).reshape(1, IN_CH)
    b2r = b2.astype(f32).reshape(1, OUT_CH)
    as2 = a2_src.astype(f32).reshape(1, OUT_CH)
    ad2 = a2_dst.astype(f32).reshape(1, OUT_CH)

    grid = (NPAD // BLK,)

    G1, A1 = pl.pallas_call(
        _prep1_body,
        grid=grid,
        in_specs=[
            _row_spec(IN_CH),
            _full_spec(IN_CH, IN_CH),
            _full_spec(IN_CH, 16),
            _full_spec(IN_CH, 16),
        ],
        out_specs=[_row_spec(D1), _row_spec(16)],
        out_shape=[
            jax.ShapeDtypeStruct((NPAD, D1), f32),
            jax.ShapeDtypeStruct((NPAD, 16), f32),
        ],
    )(xpad, W1.astype(f32), As1d, Ad1d)

    acc1 = _edge_pass_l1(G1, A1, sd1)

    G2, A2 = pl.pallas_call(
        _mid_body,
        grid=grid,
        in_specs=[
            _row_spec(D1),
            _row_spec(D1),
            _full_spec(IN_CH, OUT_CH),
            _full_spec(1, OUT_CH),
            _full_spec(1, OUT_CH),
            _full_spec(HEADS, IN_CH),
            _full_spec(1, IN_CH),
        ],
        out_specs=[_row_spec(D2), _row_spec(16)],
        out_shape=[
            jax.ShapeDtypeStruct((NPAD, D2), f32),
            jax.ShapeDtypeStruct((NPAD, 16), f32),
        ],
    )(acc1[0], acc1[1], W2.astype(f32), as2, ad2, R, b1r)

    acc2 = _edge_pass_l2(G2, A2, sd2)

    o = pl.pallas_call(
        _final_body,
        grid=grid,
        in_specs=[_row_spec(D2), _row_spec(D2), _full_spec(1, OUT_CH)],
        out_specs=_row_spec(OUT_CH),
        out_shape=jax.ShapeDtypeStruct((NPAD, OUT_CH), f32),
    )(acc2[0], acc2[1], b2r)

    return o[:N]
